# Initial kernel scaffold; baseline (speedup 1.0000x reference)
import jax
import jax.numpy as jnp
from jax.experimental import pallas as pl


def kernel(x, params):
    raise NotImplementedError("write your pallas kernel here")


# speedup vs baseline: 3.4767x; 3.4767x over previous
import jax
import jax.numpy as jnp
import numpy as np
from jax.experimental import pallas as pl
from jax.experimental.pallas import tpu as pltpu

C = 96
E = 6
GK = 11
H = 224
W = 224
HB = 8
NSTRIP = H // HB
NPIX = H * W
HPAD = 8

F32 = jnp.float32


def _gelu(x):
    return 0.5 * x * (1.0 + jax.lax.erf(x * np.float32(0.7071067811865476)))


def _full_spec(shape):
    return pl.BlockSpec(shape, lambda i: (0,) * len(shape))


def _strip_spec(hb=HB, w=W, c=C):
    return pl.BlockSpec((hb, w, c), lambda i: (i, 0, 0))


def _row(r, d):
    return r[d:d + 1, :][None]


def _ln_val(x, w2, b2):
    u = jnp.mean(x, axis=-1, keepdims=True)
    s = jnp.mean((x - u) ** 2, axis=-1, keepdims=True)
    xn = (x - u) * jax.lax.rsqrt(s + 1e-6)
    return xn * w2[None] + b2[None]


def _ln_body(x_ref, w_ref, b_ref, o_ref):
    o_ref[...] = _ln_val(x_ref[...], w_ref[...], b_ref[...])


def _layer_norm(x, w, b):
    return pl.pallas_call(
        _ln_body,
        grid=(NSTRIP,),
        in_specs=[_strip_spec(), _full_spec((1, C)), _full_spec((1, C))],
        out_specs=_strip_spec(),
        out_shape=jax.ShapeDtypeStruct((H, W, C), F32),
    )(x, w, b)


def _stem_body(xp_ref, w9_ref, b1_ref, wc_ref, bc_ref, x1_ref, k0_ref):
    i = pl.program_id(0)
    r0 = i * HB
    blk = xp_ref[pl.ds(r0, HB + 2), :, :]
    acc = jnp.zeros((HB * W, C), F32)
    for dy in range(3):
        for dx in range(3):
            sl = blk[dy:dy + HB, dx:dx + W, :].reshape(HB * W, C)
            acc += jnp.dot(sl, w9_ref[3 * dy + dx], preferred_element_type=F32)
    a = _gelu(acc + b1_ref[...])
    o = jnp.dot(a, wc_ref[...], preferred_element_type=F32) + bc_ref[...]
    x1_ref[...] = o[:, :C].reshape(HB, W, C)
    k0_ref[...] = o[:, C:].reshape(HB, W, C)


def _stem(ln1p, w9, b1, wc, bc):
    return pl.pallas_call(
        _stem_body,
        grid=(NSTRIP,),
        in_specs=[_full_spec((H + 2, W + 2, C)),
                  _full_spec((9, C, C)),
                  _full_spec((1, C)),
                  _full_spec((C, 2 * C)),
                  _full_spec((1, 2 * C))],
        out_specs=[_strip_spec(), _strip_spec()],
        out_shape=[jax.ShapeDtypeStruct((H, W, C), F32),
                   jax.ShapeDtypeStruct((H, W, C), F32)],
    )(ln1p, w9, b1, wc, bc)


def _stripes_body(xp_ref, w1_ref, b1_ref, w2_ref, b2_ref, o_ref, pool_ref):
    i = pl.program_id(0)
    r0 = i * HB
    blk = xp_ref[pl.ds(r0, HB + 2), :, :]
    s1 = jnp.zeros((HB + 2, W, C), F32)
    for d in range(3):
        s1 += blk[:, d:d + W, :] * _row(w1_ref[...], d)
    s1 = s1 + b1_ref[...][None]
    absrow = jax.lax.broadcasted_iota(jnp.int32, (HB + 2, 1, 1), 0) + (r0 - 1)
    s1 = jnp.where((absrow >= 0) & (absrow < H), s1, 0.0)
    s2 = jnp.zeros((HB, W, C), F32)
    for d in range(3):
        s2 += s1[d:d + HB, :, :] * _row(w2_ref[...], d)
    x2 = _gelu(s2 + b2_ref[...][None])
    o_ref[...] = x2
    psum = jnp.sum(x2.reshape(HB * W, C), axis=0, keepdims=True)

    @pl.when(i == 0)
    def _():
        pool_ref[...] = psum

    @pl.when(i > 0)
    def _():
        pool_ref[...] += psum


def _stripes(x1p, s1_w, s1_b, s2_w, s2_b):
    return pl.pallas_call(
        _stripes_body,
        grid=(NSTRIP,),
        in_specs=[_full_spec((H + 2, W + 2, C)),
                  _full_spec((3, C)), _full_spec((1, C)),
                  _full_spec((3, C)), _full_spec((1, C))],
        out_specs=[_strip_spec(), pl.BlockSpec((1, C), lambda i: (0, 0))],
        out_shape=[jax.ShapeDtypeStruct((H, W, C), F32),
                   jax.ShapeDtypeStruct((1, C), F32)],
    )(x1p, s1_w, s1_b, s2_w, s2_b)


def _dw4s4_ref(x_ref, w, b, hin):
    hout = hin // 4
    acc = jnp.zeros((hout, hout, C), F32)
    for iy in range(4):
        for ix in range(4):
            sl = x_ref[iy::4, ix::4, :]
            acc += sl * _row(w, 4 * iy + ix)
    return _gelu(acc + b[None])


def _cal_body(k0_ref, agg_w_ref, agg_b_ref, c1w_ref, c1b_ref, c2w_ref,
              c2b_ref, o_ref, d1_ref):
    d1_ref[...] = _dw4s4_ref(k0_ref, agg_w_ref[...], agg_b_ref[...], H)
    d2 = _dw4s4_ref(d1_ref, agg_w_ref[...], agg_b_ref[...], 56)
    z_row = jnp.zeros((1, 14, C), F32)
    d2p = jnp.concatenate([z_row, d2, z_row], axis=0)
    z_col = jnp.zeros((16, 1, C), F32)
    d2p = jnp.concatenate([z_col, d2p, z_col], axis=1)
    c = jnp.zeros((14, 14, C), F32)
    for dy in range(3):
        for dx in range(3):
            c += d2p[dy:dy + 14, dx:dx + 14, :] * _row(c1w_ref[...], 3 * dy + dx)
    c = c + c1b_ref[...][None]
    o = jnp.dot(c.reshape(196, C), c2w_ref[...], preferred_element_type=F32)
    o_ref[...] = (o + c2b_ref[...]).reshape(14, 14, C)


def _calibrate_small(k0, agg_w, agg_b, c1w, c1b, c2w, c2b):
    return pl.pallas_call(
        _cal_body,
        grid=(1,),
        in_specs=[_full_spec((H, W, C)), _full_spec((16, C)),
                  _full_spec((1, C)),
                  _full_spec((9, C)), _full_spec((1, C)),
                  _full_spec((C, C)), _full_spec((1, C))],
        out_specs=_full_spec((14, 14, C)),
        out_shape=jax.ShapeDtypeStruct((14, 14, C), F32),
        scratch_shapes=[pltpu.VMEM((56, 56, C), F32)],
    )(k0, agg_w, agg_b, c1w, c1b, c2w, c2b)


def _up_body(small_ref, k0_ref, o_ref):
    j = pl.program_id(0)
    jm = jnp.maximum(j - 1, 0)
    jp = jnp.minimum(j + 1, 13)
    prev = small_ref[jm]
    cur = small_ref[j]
    nxt = small_ref[jp]
    i8 = jax.lax.broadcasted_iota(jnp.int32, (8, 1, 1), 0).astype(F32)
    f = (i8 + 8.5) * (1.0 / 16.0)
    g = (i8 + 0.5) * (1.0 / 16.0)
    top = prev[None] * (1.0 - f) + cur[None] * f
    bot = cur[None] * (1.0 - g) + nxt[None] * g
    t = jnp.concatenate([top, bot], axis=0)
    left = jnp.concatenate([t[:, :1, :], t[:, :-1, :]], axis=1)
    right = jnp.concatenate([t[:, 1:, :], t[:, -1:, :]], axis=1)
    i8w = jax.lax.broadcasted_iota(jnp.int32, (1, 1, 8, 1), 2).astype(F32)
    fw = (i8w + 8.5) * (1.0 / 16.0)
    gw = (i8w + 0.5) * (1.0 / 16.0)
    o1 = left[:, :, None, :] * (1.0 - fw) + t[:, :, None, :] * fw
    o2 = t[:, :, None, :] * (1.0 - gw) + right[:, :, None, :] * gw
    up = jnp.concatenate([o1, o2], axis=2).reshape(16, W, C)
    o_ref[...] = k0_ref[...] + up


def _upsample_add(small, k0):
    return pl.pallas_call(
        _up_body,
        grid=(14,),
        in_specs=[_full_spec((14, 14, C)), _strip_spec(hb=16)],
        out_specs=_strip_spec(hb=16),
        out_shape=jax.ShapeDtypeStruct((H, W, C), F32),
    )(small, k0)


def _moe_body(x2_ref, k_ref, res_ref, pool_ref, rw_ref,
              w1s_ref, w2s_ref, w3s_ref, b1s_ref, b2s_ref, b3s_ref,
              pw_ref, pb_ref, o_ref,
              selw1, selw2, selw3, selb12, selb3):
    i = pl.program_id(0)

    @pl.when(i == 0)
    def _():
        pooled = pool_ref[...] * (1.0 / NPIX)
        t = pooled * rw_ref[...]
        logits = jnp.sum(t, axis=1, keepdims=True)
        m = jnp.max(logits)
        ex = jnp.exp(logits - m)
        sm = ex / jnp.sum(ex)
        idx = jax.lax.broadcasted_iota(jnp.int32, (E, 1), 0)
        g_a = jnp.max(sm)
        a = jnp.min(jnp.where(sm == g_a, idx, E))
        sm2 = jnp.where(idx == a, -1.0, sm)
        g_b = jnp.max(sm2)
        b = jnp.min(jnp.where(sm2 == g_b, idx, E))
        selw1[:, 0:HPAD] = w1s_ref[a]
        selw1[:, HPAD:] = w1s_ref[b]
        selw2[:, 0:HPAD] = w2s_ref[a]
        selw2[:, HPAD:] = w2s_ref[b]
        selw3[0:HPAD, :] = w3s_ref[a] * g_a
        selw3[HPAD:, :] = w3s_ref[b] * g_b
        selb12[0:1, 0:HPAD] = b1s_ref[a]
        selb12[0:1, HPAD:] = b1s_ref[b]
        selb12[1:2, 0:HPAD] = b2s_ref[a]
        selb12[1:2, HPAD:] = b2s_ref[b]
        selb3[...] = b3s_ref[a] * g_a + b3s_ref[b] * g_b

    x2 = x2_ref[...].reshape(HB * W, C)
    kk = k_ref[...].reshape(HB * W, C)
    hx = jnp.dot(x2, selw1[...], preferred_element_type=F32) + selb12[0:1, :]
    hk = jnp.dot(kk, selw2[...], preferred_element_type=F32) + selb12[1:2, :]
    p = hx * hk
    e = jnp.dot(p, selw3[...], preferred_element_type=F32) + selb3[...]
    moe = x2 + e
    y = jnp.dot(moe, pw_ref[...], preferred_element_type=F32) + pb_ref[...]
    o_ref[...] = y.reshape(HB, W, C) + res_ref[...]


def _moe_proj(x2, k, res, pool, rw, w1s, w2s, w3s, b1s, b2s, b3s, pw, pb):
    return pl.pallas_call(
        _moe_body,
        grid=(NSTRIP,),
        in_specs=[_strip_spec(), _strip_spec(), _strip_spec(),
                  pl.BlockSpec((1, C), lambda i: (0, 0)),
                  _full_spec((E, C)),
                  _full_spec((E, C, HPAD)), _full_spec((E, C, HPAD)),
                  _full_spec((E, HPAD, C)),
                  _full_spec((E, 1, HPAD)), _full_spec((E, 1, HPAD)),
                  _full_spec((E, 1, C)),
                  _full_spec((C, C)), _full_spec((1, C))],
        out_specs=_strip_spec(),
        out_shape=jax.ShapeDtypeStruct((H, W, C), F32),
        scratch_shapes=[pltpu.VMEM((C, 2 * HPAD), F32),
                        pltpu.VMEM((C, 2 * HPAD), F32),
                        pltpu.VMEM((2 * HPAD, C), F32),
                        pltpu.VMEM((2, 2 * HPAD), F32),
                        pltpu.VMEM((1, C), F32)],
    )(x2, k, res, pool, rw, w1s, w2s, w3s, b1s, b2s, b3s, pw, pb)


def _ln_mm_body(x_ref, nw_ref, nb_ref, w_ref, b_ref, u_ref, g_ref):
    h = _ln_val(x_ref[...], nw_ref[...], nb_ref[...])
    t = jnp.dot(h.reshape(HB * W, C), w_ref[...], preferred_element_type=F32)
    t = _gelu(t + b_ref[...])
    u_ref[...] = t[:, :C].reshape(HB, W, C)
    g_ref[...] = t[:, C:].reshape(HB, W, C)


def _ln_mm_split(x, nw, nb, w, b):
    return pl.pallas_call(
        _ln_mm_body,
        grid=(NSTRIP,),
        in_specs=[_strip_spec(), _full_spec((1, C)), _full_spec((1, C)),
                  _full_spec((C, 2 * C)), _full_spec((1, 2 * C))],
        out_specs=[_strip_spec(), _strip_spec()],
        out_shape=[jax.ShapeDtypeStruct((H, W, C), F32),
                   jax.ShapeDtypeStruct((H, W, C), F32)],
    )(x, nw, nb, w, b)


def _ffn_tail_body(u_ref, gp_ref, gw_ref, gb_ref, w2_ref, b2_ref, res_ref,
                   o_ref):
    i = pl.program_id(0)
    r0 = i * HB
    blk = gp_ref[pl.ds(r0, HB + 2), :, :]
    g2 = jnp.zeros((HB, W, C), F32)
    for dy in range(3):
        for dx in range(3):
            g2 += blk[dy:dy + HB, dx:dx + W, :] * _row(gw_ref[...], 3 * dy + dx)
    g2 = g2 + gb_ref[...][None]
    xv = (u_ref[...] * g2).reshape(HB * W, C)
    o = jnp.dot(xv, w2_ref[...], preferred_element_type=F32) + b2_ref[...]
    o_ref[...] = _gelu(o).reshape(HB, W, C) + res_ref[...]


def _ffn_tail(u, gp, gw, gb, w2, b2, res):
    return pl.pallas_call(
        _ffn_tail_body,
        grid=(NSTRIP,),
        in_specs=[_strip_spec(), _full_spec((H + 2, W + 2, C)),
                  _full_spec((9, C)), _full_spec((1, C)),
                  _full_spec((C, C)), _full_spec((1, C)),
                  _strip_spec()],
        out_specs=_strip_spec(),
        out_shape=jax.ShapeDtypeStruct((H, W, C), F32),
    )(u, gp, gw, gb, w2, b2, res)


def _smf_tail_body(qp_ref, v_ref, a1w_ref, a1b_ref, a2w_ref, a2b_ref,
                   spw_ref, spb_ref, res_ref, o_ref):
    i = pl.program_id(0)
    r0 = i * HB
    blk = qp_ref[pl.ds(r0, HB + 10), :, :]
    a1 = jnp.zeros((HB + 10, W, C), F32)
    for d in range(GK):
        a1 += blk[:, d:d + W, :] * _row(a1w_ref[...], d)
    a1 = a1 + a1b_ref[...][None]
    absrow = jax.lax.broadcasted_iota(jnp.int32, (HB + 10, 1, 1), 0) + (r0 - 5)
    a1 = jnp.where((absrow >= 0) & (absrow < H), a1, 0.0)
    a2 = jnp.zeros((HB, W, C), F32)
    for d in range(GK):
        a2 += a1[d:d + HB, :, :] * _row(a2w_ref[...], d)
    q = a2 + a2b_ref[...][None]
    qv = (q * v_ref[...]).reshape(HB * W, C)
    o = jnp.dot(qv, spw_ref[...], preferred_element_type=F32) + spb_ref[...]
    o_ref[...] = o.reshape(HB, W, C) + res_ref[...]


def _smf_tail(qp, v, a1w, a1b, a2w, a2b, spw, spb, res):
    return pl.pallas_call(
        _smf_tail_body,
        grid=(NSTRIP,),
        in_specs=[_full_spec((H + 10, W + 10, C)), _strip_spec(),
                  _full_spec((GK, C)), _full_spec((1, C)),
                  _full_spec((GK, C)), _full_spec((1, C)),
                  _full_spec((C, C)), _full_spec((1, C)),
                  _strip_spec()],
        out_specs=_strip_spec(),
        out_shape=jax.ShapeDtypeStruct((H, W, C), F32),
    )(qp, v, a1w, a1b, a2w, a2b, spw, spb, res)


def _r2(v):
    return v.reshape(1, -1)


def _prep(params):
    rme = params['rme']
    w = {}
    w['c1a_9'] = jnp.transpose(rme['c1a_w'], (2, 3, 1, 0)).reshape(9, C, C)
    w['c1a_b'] = _r2(rme['c1a_b'])
    w['c1b'] = jnp.transpose(rme['c1b_w'][:, :, 0, 0], (1, 0))
    w['c1b_b'] = _r2(rme['c1b_b'])
    w['s1'] = jnp.transpose(rme['s1_w'][:, 0, 0, :], (1, 0))
    w['s1_b'] = _r2(rme['s1_b'])
    w['s2'] = jnp.transpose(rme['s2_w'][:, 0, :, 0], (1, 0))
    w['s2_b'] = _r2(rme['s2_b'])
    w['agg'] = jnp.transpose(rme['agg_w'][:, 0].reshape(C, 16), (1, 0))
    w['agg_b'] = _r2(rme['agg_b'])
    w['cal1'] = jnp.transpose(rme['cal1_w'][:, 0].reshape(C, 9), (1, 0))
    w['cal1_b'] = _r2(rme['cal1_b'])
    w['cal2'] = jnp.transpose(rme['cal2_w'][:, :, 0, 0], (1, 0))
    w['cal2_b'] = _r2(rme['cal2_b'])
    w['router'] = rme['router_w']
    w1s, w2s, w3s, b1s, b2s, b3s = [], [], [], [], [], []
    for i in range(E):
        e = rme['experts'][i]
        pad = HPAD - (i + 2)
        w1s.append(jnp.pad(jnp.transpose(e['w1'][:, :, 0, 0], (1, 0)),
                           ((0, 0), (0, pad))))
        w2s.append(jnp.pad(jnp.transpose(e['w2'][:, :, 0, 0], (1, 0)),
                           ((0, 0), (0, pad))))
        w3s.append(jnp.pad(jnp.transpose(e['w3'][:, :, 0, 0], (1, 0)),
                           ((0, pad), (0, 0))))
        b1s.append(_r2(jnp.pad(e['b1'], (0, pad))))
        b2s.append(_r2(jnp.pad(e['b2'], (0, pad))))
        b3s.append(_r2(e['b3']))
    w['w1s'] = jnp.stack(w1s)
    w['w2s'] = jnp.stack(w2s)
    w['w3s'] = jnp.stack(w3s)
    w['b1s'] = jnp.stack(b1s)
    w['b2s'] = jnp.stack(b2s)
    w['b3s'] = jnp.stack(b3s)
    w['proj'] = jnp.transpose(rme['proj_w'][:, :, 0, 0], (1, 0))
    w['proj_b'] = _r2(rme['proj_b'])
    for name, p in (('ffn1', params['ffn1']), ('ffn2', params['ffn2'])):
        w[name + '_f1'] = jnp.transpose(p['f1_w'][:, :, 0, 0], (1, 0))
        w[name + '_f1b'] = _r2(p['f1_b'])
        w[name + '_g'] = jnp.transpose(p['g_w'][:, 0].reshape(C, 9), (1, 0))
        w[name + '_gb'] = _r2(p['g_b'])
        w[name + '_f2'] = jnp.transpose(p['f2_w'][:, :, 0, 0], (1, 0))
        w[name + '_f2b'] = _r2(p['f2_b'])
    smf = params['smf']
    w['qv'] = jnp.transpose(smf['qv_w'][:, :, 0, 0], (1, 0))
    w['qv_b'] = _r2(smf['qv_b'])
    w['a1'] = jnp.transpose(smf['a1_w'][:, 0, 0, :], (1, 0))
    w['a1_b'] = _r2(smf['a1_b'])
    w['a2'] = jnp.transpose(smf['a2_w'][:, 0, :, 0], (1, 0))
    w['a2_b'] = _r2(smf['a2_b'])
    w['sp'] = jnp.transpose(smf['sp_w'][:, :, 0, 0], (1, 0))
    w['sp_b'] = _r2(smf['sp_b'])
    return w


def _pad1(a, p=1):
    return jnp.pad(a, ((p, p), (p, p), (0, 0)))


def kernel(x, params):
    w = _prep(params)
    x3 = jnp.transpose(x[0], (1, 2, 0))

    ln1 = _layer_norm(x3, _r2(params['n1_w']), _r2(params['n1_b']))
    x1, k0 = _stem(_pad1(ln1), w['c1a_9'], w['c1a_b'], w['c1b'], w['c1b_b'])
    x2, pool = _stripes(_pad1(x1), w['s1'], w['s1_b'], w['s2'], w['s2_b'])
    small = _calibrate_small(k0, w['agg'], w['agg_b'], w['cal1'], w['cal1_b'],
                             w['cal2'], w['cal2_b'])
    k = _upsample_add(small, k0)
    y1 = _moe_proj(x2, k, x3, pool, w['router'], w['w1s'], w['w2s'], w['w3s'],
                   w['b1s'], w['b2s'], w['b3s'], w['proj'], w['proj_b'])

    u, g = _ln_mm_split(y1, _r2(params['n2_w']), _r2(params['n2_b']),
                        w['ffn1_f1'], w['ffn1_f1b'])
    y2 = _ffn_tail(u, _pad1(g), w['ffn1_g'], w['ffn1_gb'],
                   w['ffn1_f2'], w['ffn1_f2b'], y1)

    q, v = _ln_mm_split(y2, _r2(params['n3_w']), _r2(params['n3_b']),
                        w['qv'], w['qv_b'])
    y3 = _smf_tail(_pad1(q, 5), v, w['a1'], w['a1_b'], w['a2'], w['a2_b'],
                   w['sp'], w['sp_b'], y2)

    u2, g2 = _ln_mm_split(y3, _r2(params['n4_w']), _r2(params['n4_b']),
                          w['ffn2_f1'], w['ffn2_f1b'])
    y4 = _ffn_tail(u2, _pad1(g2), w['ffn2_g'], w['ffn2_gb'],
                   w['ffn2_f2'], w['ffn2_f2b'], y3)

    return jnp.transpose(y4, (2, 0, 1))[None]
